# Initial kernel scaffold; baseline (speedup 1.0000x reference)
import jax
import jax.numpy as jnp
from jax.experimental import pallas as pl


def kernel(x, edge_index, batch, W1, b1, W2, b2, W3, b3, Wf, bf):
    raise NotImplementedError("write your pallas kernel here")


# speedup vs baseline: 11.4649x; 11.4649x over previous
import functools

import jax
import jax.numpy as jnp
from jax import lax
from jax.experimental import pallas as pl
from jax.experimental.pallas import tpu as pltpu
from jax.experimental.pallas import tpu_sc as plsc

_N = 10000
_E = 320000
_D = 128
_B = 64
_DOUT = 1317
_DPAD = 1408
_NC, _NS = 2, 16
_NW = _NC * _NS
_EPT = _E // _NW
_CH = 128
_NCHUNK = -(-_EPT // _CH)
_EPAD = _NCHUNK * _CH
_NP = 10112
_STRIPE = _NP // _NS
_DUMMY = _N
_ROWBLK = 1000
_GRID = _N // _ROWBLK


def _sc_deg_body(dst_hbm, ones_hbm, zeros_hbm, out_hbm, idx_v, ones_v, acc):
    c = lax.axis_index("c")
    s = lax.axis_index("s")
    wid = c * _NS + s
    pltpu.sync_copy(zeros_hbm, acc.at[pl.ds(s * _STRIPE, _STRIPE)])
    pltpu.sync_copy(ones_hbm, ones_v)
    pltpu.sync_copy(dst_hbm.at[wid], idx_v)
    plsc.subcore_barrier()

    def body(j, carry):
        pltpu.sync_copy(ones_v, acc.at[idx_v.at[j]], add=True)
        return carry

    lax.fori_loop(0, _NCHUNK, body, 0)
    plsc.subcore_barrier()
    pltpu.sync_copy(acc.at[pl.ds(s * _STRIPE, _STRIPE)],
                    out_hbm.at[c, pl.ds(s * _STRIPE, _STRIPE)])


_sc_deg = pl.kernel(
    _sc_deg_body,
    out_type=jax.ShapeDtypeStruct((_NC, _NP, _D), jnp.float32),
    mesh=plsc.VectorSubcoreMesh(core_axis_name="c", subcore_axis_name="s"),
    scratch_types=[
        pltpu.VMEM((_NCHUNK, _CH), jnp.int32),
        pltpu.VMEM((_CH, _D), jnp.float32),
        pltpu.VMEM_SHARED((_NP, _D), jnp.float32),
    ],
)


def _sc_agg_body(hs_hbm, src_hbm, dst_hbm, zeros_hbm, out_hbm,
                 src_v, dst_v, rows_v, acc, sem):
    c = lax.axis_index("c")
    s = lax.axis_index("s")
    wid = c * _NS + s
    pltpu.sync_copy(zeros_hbm, acc.at[pl.ds(s * _STRIPE, _STRIPE)])
    pltpu.sync_copy(src_hbm.at[wid], src_v)
    pltpu.sync_copy(dst_hbm.at[wid], dst_v)
    plsc.subcore_barrier()

    def body(j, carry):
        pltpu.async_copy(hs_hbm.at[src_v.at[j]], rows_v, sem).wait()
        pltpu.sync_copy(rows_v, acc.at[dst_v.at[j]], add=True)
        return carry

    lax.fori_loop(0, _NCHUNK, body, 0)
    plsc.subcore_barrier()
    pltpu.sync_copy(acc.at[pl.ds(s * _STRIPE, _STRIPE)],
                    out_hbm.at[c, pl.ds(s * _STRIPE, _STRIPE)])


_sc_agg = pl.kernel(
    _sc_agg_body,
    out_type=jax.ShapeDtypeStruct((_NC, _NP, _D), jnp.float32),
    mesh=plsc.VectorSubcoreMesh(core_axis_name="c", subcore_axis_name="s"),
    scratch_types=[
        pltpu.VMEM((_NCHUNK, _CH), jnp.int32),
        pltpu.VMEM((_NCHUNK, _CH), jnp.int32),
        pltpu.VMEM((_CH, _D), jnp.float32),
        pltpu.VMEM_SHARED((_NP, _D), jnp.float32),
        pltpu.SemaphoreType.DMA,
    ],
)


def _dinv(deg_ref):
    return lax.rsqrt(deg_ref[0, :, 0:1] + deg_ref[1, :, 0:1] + 1.0)


def _mm_first_body(x_ref, w_ref, deg_ref, o_ref):
    o_ref[...] = jnp.dot(x_ref[...], w_ref[...],
                         preferred_element_type=jnp.float32) * _dinv(deg_ref)


def _mm_mid_body(acc_ref, hs_ref, deg_ref, b_ref, w_ref, o_ref):
    dinv = _dinv(deg_ref)
    t = acc_ref[0] + acc_ref[1] + hs_ref[...]
    xl = jnp.maximum(t * dinv + b_ref[...], 0.0)
    o_ref[...] = jnp.dot(xl, w_ref[...],
                         preferred_element_type=jnp.float32) * dinv


def _pool_body(acc_ref, hs_ref, deg_ref, b_ref, batch_ref, wf_ref, bf_ref,
               o_ref, sums, cnts):
    i = pl.program_id(0)
    dinv = _dinv(deg_ref)
    t = acc_ref[0] + acc_ref[1] + hs_ref[...]
    xl = jnp.maximum(t * dinv + b_ref[...], 0.0)
    bids = batch_ref[0, 0, :]
    oh = (bids[None, :] ==
          lax.broadcasted_iota(jnp.int32, (_B, _ROWBLK), 0)).astype(jnp.float32)

    @pl.when(i == 0)
    def _():
        sums[...] = jnp.zeros_like(sums)
        cnts[...] = jnp.zeros_like(cnts)

    sums[...] += jnp.dot(oh, xl, preferred_element_type=jnp.float32)
    cnts[...] += jnp.broadcast_to(jnp.sum(oh, axis=1, keepdims=True),
                                  (_B, _D))

    @pl.when(i == _GRID - 1)
    def _():
        pooled = sums[...] / jnp.maximum(cnts[...], 1.0)
        logits = jnp.dot(pooled, wf_ref[...],
                         preferred_element_type=jnp.float32) + bf_ref[...]
        m = jnp.max(logits, axis=1, keepdims=True)
        lse = jnp.log(jnp.sum(jnp.exp(logits - m), axis=1, keepdims=True))
        o_ref[...] = logits - m - lse


_row_spec = pl.BlockSpec((_ROWBLK, _D), lambda i: (i, 0))
_acc_spec = pl.BlockSpec((_NC, _ROWBLK, _D), lambda i: (0, i, 0))
_deg_spec = pl.BlockSpec((_NC, _ROWBLK, _D), lambda i: (0, i, 0))
_w_spec = pl.BlockSpec((_D, _D), lambda i: (0, 0))
_b_spec = pl.BlockSpec((1, _D), lambda i: (0, 0))

_mm_first = pl.pallas_call(
    _mm_first_body,
    grid=(_GRID,),
    in_specs=[_row_spec, _w_spec, _deg_spec],
    out_specs=_row_spec,
    out_shape=jax.ShapeDtypeStruct((_N, _D), jnp.float32),
)

_mm_mid = pl.pallas_call(
    _mm_mid_body,
    grid=(_GRID,),
    in_specs=[_acc_spec, _row_spec, _deg_spec, _b_spec, _w_spec],
    out_specs=_row_spec,
    out_shape=jax.ShapeDtypeStruct((_N, _D), jnp.float32),
)

_pool = pl.pallas_call(
    _pool_body,
    grid=(_GRID,),
    in_specs=[
        _acc_spec, _row_spec, _deg_spec, _b_spec,
        pl.BlockSpec((1, 1, _ROWBLK), lambda i: (i, 0, 0)),
        pl.BlockSpec((_D, _DPAD), lambda i: (0, 0)),
        pl.BlockSpec((1, _DPAD), lambda i: (0, 0)),
    ],
    out_specs=pl.BlockSpec((_B, _DPAD), lambda i: (0, 0)),
    out_shape=jax.ShapeDtypeStruct((_B, _DPAD), jnp.float32),
    scratch_shapes=[
        pltpu.VMEM((_B, _D), jnp.float32),
        pltpu.VMEM((_B, _D), jnp.float32),
    ],
)


@jax.jit
def kernel(x, edge_index, batch, W1, b1, W2, b2, W3, b3, Wf, bf):
    src = edge_index[0].astype(jnp.int32).reshape(_NW, _EPT)
    dst = edge_index[1].astype(jnp.int32).reshape(_NW, _EPT)
    pad = _EPAD - _EPT
    srcp = jnp.pad(src, ((0, 0), (0, pad))).reshape(_NW, _NCHUNK, _CH)
    dstp = jnp.pad(dst, ((0, 0), (0, pad)),
                   constant_values=_DUMMY).reshape(_NW, _NCHUNK, _CH)

    ones128 = jnp.ones((_CH, _D), jnp.float32)
    z128 = jnp.zeros((_STRIPE, _D), jnp.float32)

    degp = _sc_deg(dstp, ones128, z128)

    b1r = b1.reshape(1, _D)
    b2r = b2.reshape(1, _D)
    b3r = b3.reshape(1, _D)
    wfp = jnp.zeros((_D, _DPAD), jnp.float32).at[:, :_DOUT].set(Wf)
    bfp = jnp.full((1, _DPAD), -1e30, jnp.float32).at[0, :_DOUT].set(bf)
    batch3 = batch.astype(jnp.int32).reshape(_GRID, 1, _ROWBLK)

    hs1 = _mm_first(x, W1, degp)
    acc1 = _sc_agg(hs1, srcp, dstp, z128)
    hs2 = _mm_mid(acc1, hs1, degp, b1r, W2)
    acc2 = _sc_agg(hs2, srcp, dstp, z128)
    hs3 = _mm_mid(acc2, hs2, degp, b2r, W3)
    acc3 = _sc_agg(hs3, srcp, dstp, z128)

    outp = _pool(acc3, hs3, degp, b3r, batch3, wfp, bfp)
    return outp[:, :_DOUT]
